# Initial kernel scaffold; baseline (speedup 1.0000x reference)
import jax
import jax.numpy as jnp
from jax.experimental import pallas as pl


def kernel(x, conv1_w, conv1_sc, conv1_sh, conv2_w, conv2_sc, conv2_sh, conv3_w, conv3_sc, conv3_sh, fc1_w, fc1_sc, fc1_sh, fc2_w, fc2_sc, fc2_sh):
    raise NotImplementedError("write your pallas kernel here")


# speedup vs baseline: 2.7287x; 2.7287x over previous
import jax
import jax.numpy as jnp
from jax.experimental import pallas as pl
from jax.experimental.pallas import tpu as pltpu

_B = 16


def _rep(shape):
    zeros = (0,) * len(shape)
    return pl.BlockSpec(shape, lambda *_: zeros)


def _pool_h(a, axis_len):
    B = a.shape[0]
    H = axis_len
    tail = jnp.zeros((B, 1) + a.shape[2:], a.dtype)
    ys = jnp.concatenate([a[:, 1:], tail], axis=1)
    ys = ys.reshape((B, H // 2, 2) + a.shape[2:])
    A = jnp.max(ys, axis=2)
    Ap = jnp.concatenate([a[:, 0:1], A[:, : H // 2 - 1]], axis=1)
    return jnp.maximum(Ap, A)


def _pool_w(a, axis_len):
    B, H = a.shape[0], a.shape[1]
    W = axis_len
    tail = jnp.zeros((B, H, 1) + a.shape[3:], a.dtype)
    ys = jnp.concatenate([a[:, :, 1:], tail], axis=2)
    ys = ys.reshape((B, H, W // 2, 2) + a.shape[3:])
    A = jnp.max(ys, axis=3)
    Ap = jnp.concatenate([a[:, :, 0:1], A[:, :, : W // 2 - 1]], axis=2)
    return jnp.maximum(Ap, A)


def _conv_trunk_kernel(x_ref, m1_ref, sc1_ref, sh1_ref, w2_ref, sc2_ref, sh2_ref,
                       w3_ref, sc3_ref, sh3_ref, fc3_ref):
    B = _B
    bf16, f32 = jnp.bfloat16, jnp.float32

    xb = x_ref[...]
    zw = jnp.zeros((B, 64, 2), bf16)
    xp = jnp.concatenate([zw, xb, zw], axis=2)
    zh = jnp.zeros((B, 2, 68), bf16)
    xp = jnp.concatenate([zh, xp, zh], axis=1)
    x5 = jnp.concatenate([xp[:, kh:kh + 64, :] for kh in range(5)], axis=2)
    y1 = jnp.dot(x5.reshape(B * 64, 340), m1_ref[...],
                 preferred_element_type=f32)
    a1 = jnp.maximum(y1 * sc1_ref[...] + sh1_ref[...], 0.0).astype(bf16)
    a1 = a1.reshape(B, 64, 1024)

    mh = _pool_h(a1, 64)
    ev, od = mh[:, :, :512], mh[:, :, 512:]
    z16 = jnp.zeros((B, 32, 16), bf16)
    odp = jnp.concatenate([z16, od[:, :, :496]], axis=2)
    evn = jnp.concatenate([ev[:, :, 16:], z16], axis=2)
    act1 = jnp.maximum(jnp.maximum(ev, od), jnp.maximum(odp, evn))
    act1 = act1.reshape(B, 32, 32, 16)

    zw = jnp.zeros((B, 32, 2, 16), bf16)
    t = jnp.concatenate([zw, act1, zw], axis=2)
    zh = jnp.zeros((B, 2, 36, 16), bf16)
    a1p = jnp.concatenate([zh, t, zh], axis=1)
    col2 = jnp.concatenate([a1p[:, :, kw:kw + 32, :] for kw in range(5)], axis=3)
    z2 = jnp.dot(col2.reshape(B * 36 * 32, 80), w2_ref[...],
                 preferred_element_type=f32).reshape(B, 36, 32, 160)
    y2 = z2[:, 0:32, :, 0:32]
    for kh in range(1, 5):
        y2 = y2 + z2[:, kh:kh + 32, :, 32 * kh:32 * kh + 32]
    a2 = jnp.maximum(y2 * sc2_ref[...].reshape(1, 1, 1, 32)
                     + sh2_ref[...].reshape(1, 1, 1, 32), 0.0).astype(bf16)

    act2 = _pool_w(_pool_h(a2, 32), 32)

    zw = jnp.zeros((B, 16, 2, 32), bf16)
    t = jnp.concatenate([zw, act2, zw], axis=2)
    zh = jnp.zeros((B, 2, 20, 32), bf16)
    a2p = jnp.concatenate([zh, t, zh], axis=1)
    col3 = jnp.concatenate([a2p[:, :, kw:kw + 16, :] for kw in range(5)], axis=3)
    z3 = jnp.dot(col3.reshape(B * 20 * 16, 160), w3_ref[...],
                 preferred_element_type=f32).reshape(B, 20, 16, 320)
    y3 = z3[:, 0:16, :, 0:64]
    for kh in range(1, 5):
        y3 = y3 + z3[:, kh:kh + 16, :, 64 * kh:64 * kh + 64]
    a3 = jnp.maximum(y3 * sc3_ref[...].reshape(1, 1, 1, 64)
                     + sh3_ref[...].reshape(1, 1, 1, 64), 0.0)

    act3 = _pool_w(_pool_h(a3, 16), 16)

    t = act3.reshape(B, 64, 64)
    fc3_ref[...] = jnp.transpose(t, (0, 2, 1)).reshape(B, 4096)


def _fc_kernel(x_ref, w1_ref, sc1_ref, sh1_ref, w2_ref, sc2_ref, sh2_ref,
               f1_ref, f2_ref):
    f32 = jnp.float32
    xb = x_ref[...].astype(jnp.bfloat16)
    z1 = jnp.dot(xb, w1_ref[...], preferred_element_type=f32)
    f1 = jnp.maximum(z1 * sc1_ref[...] + sh1_ref[...], 0.0)
    f1_ref[...] = f1
    z2 = jnp.dot(f1.astype(jnp.bfloat16), w2_ref[...], preferred_element_type=f32)
    f2_ref[...] = jnp.maximum(z2 * sc2_ref[...] + sh2_ref[...], 0.0)


def kernel(x, conv1_w, conv1_sc, conv1_sh, conv2_w, conv2_sc, conv2_sh,
           conv3_w, conv3_sc, conv3_sh, fc1_w, fc1_sc, fc1_sh,
           fc2_w, fc2_sc, fc2_sh):
    bf16, f32 = jnp.bfloat16, jnp.float32
    N = x.shape[0]
    B = _B
    xb = x.reshape(N, 64, 64).astype(bf16)

    m1 = jnp.zeros((5, 68, 64, 16), f32)
    ar = jnp.arange(64)
    for kw in range(5):
        m1 = m1.at[:, ar + kw, ar, :].set(
            jnp.broadcast_to(conv1_w[:, kw, None, :], (5, 64, 16)))
    m1 = (m1.reshape(5, 68, 32, 2, 16).transpose(0, 1, 3, 2, 4)
          .reshape(340, 1024).astype(bf16))
    sc1b = jnp.tile(conv1_sc.reshape(16), 64).reshape(1, 1024)
    sh1b = jnp.tile(conv1_sh.reshape(16), 64).reshape(1, 1024)

    w2b = conv2_w.transpose(1, 0, 2).reshape(80, 160).astype(bf16)
    w3b = conv3_w.transpose(1, 0, 2).reshape(160, 320).astype(bf16)

    fmap = pl.pallas_call(
        _conv_trunk_kernel,
        out_shape=jax.ShapeDtypeStruct((N, 4096), f32),
        grid=(N // B,),
        in_specs=[
            pl.BlockSpec((B, 64, 64), lambda b: (b, 0, 0)),
            _rep(m1.shape), _rep(sc1b.shape), _rep(sh1b.shape),
            _rep(w2b.shape), _rep(conv2_sc.shape), _rep(conv2_sh.shape),
            _rep(w3b.shape), _rep(conv3_sc.shape), _rep(conv3_sh.shape),
        ],
        out_specs=pl.BlockSpec((B, 4096), lambda b: (b, 0)),
        compiler_params=pltpu.CompilerParams(
            dimension_semantics=("parallel",)),
    )(xb, m1, sc1b, sh1b, w2b, conv2_sc, conv2_sh, w3b, conv3_sc, conv3_sh)

    w1f = fc1_w.astype(bf16)
    w2f = fc2_w.astype(bf16)
    f_fc1, f_fc2 = pl.pallas_call(
        _fc_kernel,
        out_shape=(jax.ShapeDtypeStruct((N, 512), f32),
                   jax.ShapeDtypeStruct((N, 256), f32)),
        grid=(2,),
        in_specs=[
            pl.BlockSpec((N // 2, 4096), lambda b: (b, 0)),
            _rep(w1f.shape), _rep(fc1_sc.shape), _rep(fc1_sh.shape),
            _rep(w2f.shape), _rep(fc2_sc.shape), _rep(fc2_sh.shape),
        ],
        out_specs=(pl.BlockSpec((N // 2, 512), lambda b: (b, 0)),
                   pl.BlockSpec((N // 2, 256), lambda b: (b, 0))),
        compiler_params=pltpu.CompilerParams(
            dimension_semantics=("parallel",)),
    )(fmap, w1f, fc1_sc, fc1_sh, w2f, fc2_sc, fc2_sh)

    return {"f_conv3": fmap, "f_fc1": f_fc1, "f_fc2": f_fc2}


# speedup vs baseline: 12.0888x; 4.4303x over previous
import jax
import jax.numpy as jnp
from jax.experimental import pallas as pl
from jax.experimental.pallas import tpu as pltpu

_B = 16


def _rep(shape):
    zeros = (0,) * len(shape)
    return pl.BlockSpec(shape, lambda *_: zeros)


def _pool_h(a):
    H = a.shape[0]
    r = a.reshape((H // 2, 2) + a.shape[1:])
    e, o = r[:, 0], r[:, 1]
    z = jnp.zeros_like(e[0:1])
    op = jnp.concatenate([z, o[:-1]], axis=0)
    en = jnp.concatenate([e[1:], z], axis=0)
    return jnp.maximum(jnp.maximum(e, o), jnp.maximum(op, en))


def _pool_w(a, step):
    ev, od = a[..., :512], a[..., 512:]
    z = jnp.zeros(a.shape[:-1] + (step,), a.dtype)
    odp = jnp.concatenate([z, od[..., :512 - step]], axis=-1)
    evn = jnp.concatenate([ev[..., step:], z], axis=-1)
    return jnp.maximum(jnp.maximum(ev, od), jnp.maximum(odp, evn))


def _conv_trunk_kernel(x_ref, m1_ref, sc1_ref, sh1_ref, m2_ref, sc2_ref, sh2_ref,
                       m3_ref, sc3_ref, sh3_ref, fc3_ref):
    B = _B
    bf16, f32 = jnp.bfloat16, jnp.float32

    xb = x_ref[...]
    zw = jnp.zeros((64, B, 2), bf16)
    xp = jnp.concatenate([zw, xb, zw], axis=2)
    zh = jnp.zeros((2, B, 68), bf16)
    xp = jnp.concatenate([zh, xp, zh], axis=0)
    x5 = jnp.concatenate([xp[kh:kh + 64] for kh in range(5)], axis=2)
    y1 = jnp.dot(x5.reshape(64 * B, 340), m1_ref[...],
                 preferred_element_type=f32)
    a1 = jnp.maximum(y1 * sc1_ref[...] + sh1_ref[...], 0.0).astype(bf16)
    a1 = a1.reshape(64, B, 1024)

    act1 = _pool_w(_pool_h(a1), 16)

    zw = jnp.zeros((32, B, 32), bf16)
    t = jnp.concatenate([zw, act1, zw], axis=2)
    zh = jnp.zeros((2, B, 576), bf16)
    a1p = jnp.concatenate([zh, t, zh], axis=0)
    y2 = jnp.dot(a1p[0:32].reshape(32 * B, 576), m2_ref[0],
                 preferred_element_type=f32)
    for kh in range(1, 5):
        y2 = y2 + jnp.dot(a1p[kh:kh + 32].reshape(32 * B, 576), m2_ref[kh],
                          preferred_element_type=f32)
    a2 = jnp.maximum(y2 * sc2_ref[...] + sh2_ref[...], 0.0).astype(bf16)
    a2 = a2.reshape(32, B, 1024)

    act2 = _pool_w(_pool_h(a2), 32)

    zw = jnp.zeros((16, B, 64), bf16)
    t = jnp.concatenate([zw, act2, zw], axis=2)
    zh = jnp.zeros((2, B, 640), bf16)
    a2p = jnp.concatenate([zh, t, zh], axis=0)
    y3 = jnp.dot(a2p[0:16].reshape(16 * B, 640), m3_ref[0],
                 preferred_element_type=f32)
    for kh in range(1, 5):
        y3 = y3 + jnp.dot(a2p[kh:kh + 16].reshape(16 * B, 640), m3_ref[kh],
                          preferred_element_type=f32)
    a3 = jnp.maximum(y3 * sc3_ref[...] + sh3_ref[...], 0.0)
    a3 = a3.reshape(16, B, 1024)

    fc3_ref[...] = _pool_w(_pool_h(a3), 64)


def _fc_kernel(x_ref, w1_ref, sc1_ref, sh1_ref, w2_ref, sc2_ref, sh2_ref,
               f1_ref, f2_ref):
    f32 = jnp.float32
    z1 = jnp.dot(x_ref[0].astype(jnp.bfloat16), w1_ref[0],
                 preferred_element_type=f32)
    for h in range(1, 8):
        z1 = z1 + jnp.dot(x_ref[h].astype(jnp.bfloat16), w1_ref[h],
                          preferred_element_type=f32)
    f1 = jnp.maximum(z1 * sc1_ref[...] + sh1_ref[...], 0.0)
    f1_ref[...] = f1
    z2 = jnp.dot(f1.astype(jnp.bfloat16), w2_ref[...], preferred_element_type=f32)
    f2_ref[...] = jnp.maximum(z2 * sc2_ref[...] + sh2_ref[...], 0.0)


def _band_weight(w_packed, K, Cin, Cout, Wout):
    Wpad = Wout + 4
    w4 = w_packed.reshape(K, K, Cin, Cout)
    eye = jnp.stack([jnp.eye(Wpad, Wout, -kw, dtype=w_packed.dtype)
                     for kw in range(K)])
    m = jnp.einsum('qvw,kqic->kviwc', eye, w4)
    m = m.reshape(K, Wpad, Cin, Wout // 2, 2, Cout)
    m = m.transpose(0, 1, 2, 4, 3, 5)
    return m.reshape(K, Wpad * Cin, 1024).astype(jnp.bfloat16)


def kernel(x, conv1_w, conv1_sc, conv1_sh, conv2_w, conv2_sc, conv2_sh,
           conv3_w, conv3_sc, conv3_sh, fc1_w, fc1_sc, fc1_sh,
           fc2_w, fc2_sc, fc2_sh):
    bf16, f32 = jnp.bfloat16, jnp.float32
    N = x.shape[0]
    B = _B
    xt = jnp.transpose(x.reshape(N, 64, 64).astype(bf16), (1, 0, 2))

    m1 = _band_weight(conv1_w, 5, 1, 16, 64).reshape(340, 1024)
    m2 = _band_weight(conv2_w, 5, 16, 32, 32)
    m3 = _band_weight(conv3_w, 5, 32, 64, 16)

    sc1b = jnp.tile(conv1_sc.reshape(16), 64).reshape(1, 1024)
    sh1b = jnp.tile(conv1_sh.reshape(16), 64).reshape(1, 1024)
    sc2b = jnp.tile(conv2_sc.reshape(32), 32).reshape(1, 1024)
    sh2b = jnp.tile(conv2_sh.reshape(32), 32).reshape(1, 1024)
    sc3b = jnp.tile(conv3_sc.reshape(64), 16).reshape(1, 1024)
    sh3b = jnp.tile(conv3_sh.reshape(64), 16).reshape(1, 1024)

    fmap_raw = pl.pallas_call(
        _conv_trunk_kernel,
        out_shape=jax.ShapeDtypeStruct((8, N, 512), f32),
        grid=(N // B,),
        in_specs=[
            pl.BlockSpec((64, B, 64), lambda b: (0, b, 0)),
            _rep(m1.shape), _rep(sc1b.shape), _rep(sh1b.shape),
            _rep(m2.shape), _rep(sc2b.shape), _rep(sh2b.shape),
            _rep(m3.shape), _rep(sc3b.shape), _rep(sh3b.shape),
        ],
        out_specs=pl.BlockSpec((8, B, 512), lambda b: (0, b, 0)),
        compiler_params=pltpu.CompilerParams(
            dimension_semantics=("parallel",)),
    )(xt, m1, sc1b, sh1b, m2, sc2b, sh2b, m3, sc3b, sh3b)

    f_conv3 = (fmap_raw.reshape(8, N, 8, 64).transpose(1, 3, 0, 2)
               .reshape(N, 4096))

    w1f = (fc1_w.reshape(64, 8, 8, 512).transpose(1, 2, 0, 3)
           .reshape(8, 512, 512).astype(bf16))
    w2f = fc2_w.astype(bf16)
    f_fc1, f_fc2 = pl.pallas_call(
        _fc_kernel,
        out_shape=(jax.ShapeDtypeStruct((N, 512), f32),
                   jax.ShapeDtypeStruct((N, 256), f32)),
        grid=(2,),
        in_specs=[
            pl.BlockSpec((8, N // 2, 512), lambda b: (0, b, 0)),
            _rep(w1f.shape), _rep(fc1_sc.shape), _rep(fc1_sh.shape),
            _rep(w2f.shape), _rep(fc2_sc.shape), _rep(fc2_sh.shape),
        ],
        out_specs=(pl.BlockSpec((N // 2, 512), lambda b: (b, 0)),
                   pl.BlockSpec((N // 2, 256), lambda b: (b, 0))),
        compiler_params=pltpu.CompilerParams(
            dimension_semantics=("parallel",)),
    )(fmap_raw, w1f, fc1_sc, fc1_sh, w2f, fc2_sc, fc2_sh)

    return {"f_conv3": f_conv3, "f_fc1": f_fc1, "f_fc2": f_fc2}


# speedup vs baseline: 12.4612x; 1.0308x over previous
import jax
import jax.numpy as jnp
from jax.experimental import pallas as pl
from jax.experimental.pallas import tpu as pltpu

_B = 32
_BH = 16


def _rep(shape):
    zeros = (0,) * len(shape)
    return pl.BlockSpec(shape, lambda *_: zeros)


def _pool_h(a):
    H = a.shape[0]
    r = a.reshape((H // 2, 2) + a.shape[1:])
    e, o = r[:, 0], r[:, 1]
    z = jnp.zeros_like(e[0:1])
    op = jnp.concatenate([z, o[:-1]], axis=0)
    en = jnp.concatenate([e[1:], z], axis=0)
    return jnp.maximum(jnp.maximum(e, o), jnp.maximum(op, en))


def _pool_w(a, step):
    ev, od = a[..., :512], a[..., 512:]
    z = jnp.zeros(a.shape[:-1] + (step,), a.dtype)
    odp = jnp.concatenate([z, od[..., :512 - step]], axis=-1)
    evn = jnp.concatenate([ev[..., step:], z], axis=-1)
    return jnp.maximum(jnp.maximum(ev, od), jnp.maximum(odp, evn))


def _trunk_half(xb, m1_ref, sc1_ref, sh1_ref, m2_ref, sc2_ref, sh2_ref,
                m3_ref, sc3_ref, sh3_ref):
    B = _BH
    bf16, f32 = jnp.bfloat16, jnp.float32

    zw = jnp.zeros((64, B, 2), bf16)
    xp = jnp.concatenate([zw, xb, zw], axis=2)
    zh = jnp.zeros((2, B, 68), bf16)
    xp = jnp.concatenate([zh, xp, zh], axis=0)
    x5 = jnp.concatenate([xp[kh:kh + 64] for kh in range(5)], axis=2)
    y1 = jnp.dot(x5.reshape(64 * B, 340), m1_ref[...],
                 preferred_element_type=f32)
    a1 = jnp.maximum(y1 * sc1_ref[...] + sh1_ref[...], 0.0).astype(bf16)
    a1 = a1.reshape(64, B, 1024)

    act1 = _pool_w(_pool_h(a1), 16)

    zw = jnp.zeros((32, B, 32), bf16)
    t = jnp.concatenate([zw, act1, zw], axis=2)
    zh = jnp.zeros((2, B, 576), bf16)
    a1p = jnp.concatenate([zh, t, zh], axis=0)
    y2 = jnp.dot(a1p[0:32].reshape(32 * B, 576), m2_ref[0],
                 preferred_element_type=f32)
    for kh in range(1, 5):
        y2 = y2 + jnp.dot(a1p[kh:kh + 32].reshape(32 * B, 576), m2_ref[kh],
                          preferred_element_type=f32)
    a2 = jnp.maximum(y2 * sc2_ref[...] + sh2_ref[...], 0.0).astype(bf16)
    a2 = a2.reshape(32, B, 1024)

    act2 = _pool_w(_pool_h(a2), 32)

    zw = jnp.zeros((16, B, 64), bf16)
    t = jnp.concatenate([zw, act2, zw], axis=2)
    zh = jnp.zeros((2, B, 640), bf16)
    a2p = jnp.concatenate([zh, t, zh], axis=0)
    y3 = jnp.dot(a2p[0:16].reshape(16 * B, 640), m3_ref[0],
                 preferred_element_type=f32)
    for kh in range(1, 5):
        y3 = y3 + jnp.dot(a2p[kh:kh + 16].reshape(16 * B, 640), m3_ref[kh],
                          preferred_element_type=f32)
    a3 = jnp.maximum(y3 * sc3_ref[...] + sh3_ref[...], 0.0)
    a3 = a3.reshape(16, B, 1024)

    return _pool_w(_pool_h(a3), 64)


def _conv_trunk_kernel(x_ref, m1_ref, sc1_ref, sh1_ref, m2_ref, sc2_ref, sh2_ref,
                       m3_ref, sc3_ref, sh3_ref, fc3_ref):
    args = (m1_ref, sc1_ref, sh1_ref, m2_ref, sc2_ref, sh2_ref,
            m3_ref, sc3_ref, sh3_ref)
    fc3_ref[:, 0:_BH, :] = _trunk_half(x_ref[:, 0:_BH, :], *args)
    fc3_ref[:, _BH:_B, :] = _trunk_half(x_ref[:, _BH:_B, :], *args)


def _fc_kernel(x_ref, w1_ref, sc1_ref, sh1_ref, w2_ref, sc2_ref, sh2_ref,
               f1_ref, f2_ref):
    f32 = jnp.float32
    z1 = jnp.dot(x_ref[0].astype(jnp.bfloat16), w1_ref[0],
                 preferred_element_type=f32)
    for h in range(1, 8):
        z1 = z1 + jnp.dot(x_ref[h].astype(jnp.bfloat16), w1_ref[h],
                          preferred_element_type=f32)
    f1 = jnp.maximum(z1 * sc1_ref[...] + sh1_ref[...], 0.0)
    f1_ref[...] = f1
    z2 = jnp.dot(f1.astype(jnp.bfloat16), w2_ref[...], preferred_element_type=f32)
    f2_ref[...] = jnp.maximum(z2 * sc2_ref[...] + sh2_ref[...], 0.0)


def _band_weight(w_packed, K, Cin, Cout, Wout):
    Wpad = Wout + 4
    w4 = w_packed.reshape(K, K, Cin, Cout)
    eye = jnp.stack([jnp.eye(Wpad, Wout, -kw, dtype=w_packed.dtype)
                     for kw in range(K)])
    m = jnp.einsum('qvw,kqic->kviwc', eye, w4)
    m = m.reshape(K, Wpad, Cin, Wout // 2, 2, Cout)
    m = m.transpose(0, 1, 2, 4, 3, 5)
    return m.reshape(K, Wpad * Cin, 1024).astype(jnp.bfloat16)


def kernel(x, conv1_w, conv1_sc, conv1_sh, conv2_w, conv2_sc, conv2_sh,
           conv3_w, conv3_sc, conv3_sh, fc1_w, fc1_sc, fc1_sh,
           fc2_w, fc2_sc, fc2_sh):
    bf16, f32 = jnp.bfloat16, jnp.float32
    N = x.shape[0]
    B = _B
    xt = jnp.transpose(x.reshape(N, 64, 64).astype(bf16), (1, 0, 2))

    m1 = _band_weight(conv1_w, 5, 1, 16, 64).reshape(340, 1024)
    m2 = _band_weight(conv2_w, 5, 16, 32, 32)
    m3 = _band_weight(conv3_w, 5, 32, 64, 16)

    sc1b = jnp.tile(conv1_sc.reshape(16), 64).reshape(1, 1024)
    sh1b = jnp.tile(conv1_sh.reshape(16), 64).reshape(1, 1024)
    sc2b = jnp.tile(conv2_sc.reshape(32), 32).reshape(1, 1024)
    sh2b = jnp.tile(conv2_sh.reshape(32), 32).reshape(1, 1024)
    sc3b = jnp.tile(conv3_sc.reshape(64), 16).reshape(1, 1024)
    sh3b = jnp.tile(conv3_sh.reshape(64), 16).reshape(1, 1024)

    fmap_raw = pl.pallas_call(
        _conv_trunk_kernel,
        out_shape=jax.ShapeDtypeStruct((8, N, 512), f32),
        grid=(N // B,),
        in_specs=[
            pl.BlockSpec((64, B, 64), lambda b: (0, b, 0)),
            _rep(m1.shape), _rep(sc1b.shape), _rep(sh1b.shape),
            _rep(m2.shape), _rep(sc2b.shape), _rep(sh2b.shape),
            _rep(m3.shape), _rep(sc3b.shape), _rep(sh3b.shape),
        ],
        out_specs=pl.BlockSpec((8, B, 512), lambda b: (0, b, 0)),
        compiler_params=pltpu.CompilerParams(
            dimension_semantics=("parallel",)),
    )(xt, m1, sc1b, sh1b, m2, sc2b, sh2b, m3, sc3b, sh3b)

    f_conv3 = (fmap_raw.reshape(8, N, 8, 64).transpose(1, 3, 0, 2)
               .reshape(N, 4096))

    w1f = (fc1_w.reshape(64, 8, 8, 512).transpose(1, 2, 0, 3)
           .reshape(8, 512, 512).astype(bf16))
    w2f = fc2_w.astype(bf16)
    f_fc1, f_fc2 = pl.pallas_call(
        _fc_kernel,
        out_shape=(jax.ShapeDtypeStruct((N, 512), f32),
                   jax.ShapeDtypeStruct((N, 256), f32)),
        grid=(2,),
        in_specs=[
            pl.BlockSpec((8, N // 2, 512), lambda b: (0, b, 0)),
            _rep(w1f.shape), _rep(fc1_sc.shape), _rep(fc1_sh.shape),
            _rep(w2f.shape), _rep(fc2_sc.shape), _rep(fc2_sh.shape),
        ],
        out_specs=(pl.BlockSpec((N // 2, 512), lambda b: (b, 0)),
                   pl.BlockSpec((N // 2, 256), lambda b: (b, 0))),
        compiler_params=pltpu.CompilerParams(
            dimension_semantics=("parallel",)),
    )(fmap_raw, w1f, fc1_sc, fc1_sh, w2f, fc2_sc, fc2_sh)

    return {"f_conv3": f_conv3, "f_fc1": f_fc1, "f_fc2": f_fc2}


# speedup vs baseline: 23.5218x; 1.8876x over previous
import jax
import jax.numpy as jnp
from jax.experimental import pallas as pl
from jax.experimental.pallas import tpu as pltpu

_B = 32
_BH = 16


def _rep(shape):
    zeros = (0,) * len(shape)
    return pl.BlockSpec(shape, lambda *_: zeros)


def _pool_h(a):
    H = a.shape[0]
    r = a.reshape((H // 2, 2) + a.shape[1:])
    e, o = r[:, 0], r[:, 1]
    z = jnp.zeros_like(e[0:1])
    op = jnp.concatenate([z, o[:-1]], axis=0)
    en = jnp.concatenate([e[1:], z], axis=0)
    return jnp.maximum(jnp.maximum(e, o), jnp.maximum(op, en))


def _pool_w(a, step):
    ev, od = a[..., :512], a[..., 512:]
    z = jnp.zeros(a.shape[:-1] + (step,), a.dtype)
    odp = jnp.concatenate([z, od[..., :512 - step]], axis=-1)
    evn = jnp.concatenate([ev[..., step:], z], axis=-1)
    return jnp.maximum(jnp.maximum(ev, od), jnp.maximum(odp, evn))


def _trunk_half(xb, m1_ref, sc1_ref, sh1_ref, m2_ref, sc2_ref, sh2_ref,
                m3_ref, sc3_ref, sh3_ref):
    B = _BH
    bf16, f32 = jnp.bfloat16, jnp.float32

    zw = jnp.zeros((64, B, 2), bf16)
    xp = jnp.concatenate([zw, xb, zw], axis=2)
    zh = jnp.zeros((2, B, 68), bf16)
    xp = jnp.concatenate([zh, xp, zh], axis=0)
    x5 = jnp.concatenate([xp[kh:kh + 64] for kh in range(5)], axis=2)
    y1 = jnp.dot(x5.reshape(64 * B, 340), m1_ref[...],
                 preferred_element_type=f32)
    a1 = jnp.maximum(y1 * sc1_ref[...] + sh1_ref[...], 0.0).astype(bf16)
    a1 = a1.reshape(64, B, 1024)

    act1 = _pool_w(_pool_h(a1), 16)

    zw = jnp.zeros((32, B, 32), bf16)
    t = jnp.concatenate([zw, act1, zw], axis=2)
    zh = jnp.zeros((2, B, 576), bf16)
    a1p = jnp.concatenate([zh, t, zh], axis=0)
    yg = []
    for g in range(4):
        acc = jnp.dot(a1p[0:32, :, 128 * g:128 * g + 192].reshape(32 * B, 192),
                      m2_ref[0], preferred_element_type=f32)
        for kh in range(1, 5):
            acc = acc + jnp.dot(
                a1p[kh:kh + 32, :, 128 * g:128 * g + 192].reshape(32 * B, 192),
                m2_ref[kh], preferred_element_type=f32)
        yg.append(acc)
    y2 = jnp.concatenate([y[:, :128] for y in yg]
                         + [y[:, 128:] for y in yg], axis=1)
    a2 = jnp.maximum(y2 * sc2_ref[...] + sh2_ref[...], 0.0).astype(bf16)
    a2 = a2.reshape(32, B, 1024)

    act2 = _pool_w(_pool_h(a2), 32)

    zw = jnp.zeros((16, B, 64), bf16)
    t = jnp.concatenate([zw, act2, zw], axis=2)
    zh = jnp.zeros((2, B, 640), bf16)
    a2p = jnp.concatenate([zh, t, zh], axis=0)
    yg = []
    for g in range(4):
        acc = jnp.dot(a2p[0:16, :, 128 * g:128 * g + 256].reshape(16 * B, 256),
                      m3_ref[0], preferred_element_type=f32)
        for kh in range(1, 5):
            acc = acc + jnp.dot(
                a2p[kh:kh + 16, :, 128 * g:128 * g + 256].reshape(16 * B, 256),
                m3_ref[kh], preferred_element_type=f32)
        yg.append(acc)
    y3 = jnp.concatenate([y[:, :128] for y in yg]
                         + [y[:, 128:] for y in yg], axis=1)
    a3 = jnp.maximum(y3 * sc3_ref[...] + sh3_ref[...], 0.0)
    a3 = a3.reshape(16, B, 1024)

    return _pool_w(_pool_h(a3), 64)


def _conv_trunk_kernel(x_ref, m1_ref, sc1_ref, sh1_ref, m2_ref, sc2_ref, sh2_ref,
                       m3_ref, sc3_ref, sh3_ref, fc3_ref):
    args = (m1_ref, sc1_ref, sh1_ref, m2_ref, sc2_ref, sh2_ref,
            m3_ref, sc3_ref, sh3_ref)
    fc3_ref[:, 0:_BH, :] = _trunk_half(x_ref[:, 0:_BH, :], *args)
    fc3_ref[:, _BH:_B, :] = _trunk_half(x_ref[:, _BH:_B, :], *args)


def _fc_kernel(x_ref, w1_ref, sc1_ref, sh1_ref, w2_ref, sc2_ref, sh2_ref,
               f1_ref, f2_ref):
    f32 = jnp.float32
    z1 = jnp.dot(x_ref[0].astype(jnp.bfloat16), w1_ref[0],
                 preferred_element_type=f32)
    for h in range(1, 8):
        z1 = z1 + jnp.dot(x_ref[h].astype(jnp.bfloat16), w1_ref[h],
                          preferred_element_type=f32)
    f1 = jnp.maximum(z1 * sc1_ref[...] + sh1_ref[...], 0.0)
    f1_ref[...] = f1
    z2 = jnp.dot(f1.astype(jnp.bfloat16), w2_ref[...], preferred_element_type=f32)
    f2_ref[...] = jnp.maximum(z2 * sc2_ref[...] + sh2_ref[...], 0.0)


def _band_weight(w_packed, K, Cin, Cout, Wout):
    Wpad = Wout + K - 1
    w4 = w_packed.reshape(K, K, Cin, Cout)
    eye = jnp.stack([jnp.eye(Wpad, Wout, -kw, dtype=w_packed.dtype)
                     for kw in range(K)])
    m = jnp.einsum('qvw,kqic->kviwc', eye, w4)
    m = m.reshape(K, Wpad, Cin, Wout // 2, 2, Cout)
    m = m.transpose(0, 1, 2, 4, 3, 5)
    return m.reshape(K, Wpad * Cin, Wout * Cout).astype(jnp.bfloat16)


def kernel(x, conv1_w, conv1_sc, conv1_sh, conv2_w, conv2_sc, conv2_sh,
           conv3_w, conv3_sc, conv3_sh, fc1_w, fc1_sc, fc1_sh,
           fc2_w, fc2_sc, fc2_sh):
    bf16, f32 = jnp.bfloat16, jnp.float32
    N = x.shape[0]
    B = _B
    xt = jnp.transpose(x.reshape(N, 64, 64).astype(bf16), (1, 0, 2))

    m1 = _band_weight(conv1_w, 5, 1, 16, 64).reshape(340, 1024)
    m2 = _band_weight(conv2_w, 5, 16, 32, 8)
    m3 = _band_weight(conv3_w, 5, 32, 64, 4)

    sc1b = jnp.tile(conv1_sc.reshape(16), 64).reshape(1, 1024)
    sh1b = jnp.tile(conv1_sh.reshape(16), 64).reshape(1, 1024)
    sc2b = jnp.tile(conv2_sc.reshape(32), 32).reshape(1, 1024)
    sh2b = jnp.tile(conv2_sh.reshape(32), 32).reshape(1, 1024)
    sc3b = jnp.tile(conv3_sc.reshape(64), 16).reshape(1, 1024)
    sh3b = jnp.tile(conv3_sh.reshape(64), 16).reshape(1, 1024)

    fmap_raw = pl.pallas_call(
        _conv_trunk_kernel,
        out_shape=jax.ShapeDtypeStruct((8, N, 512), f32),
        grid=(N // B,),
        in_specs=[
            pl.BlockSpec((64, B, 64), lambda b: (0, b, 0)),
            _rep(m1.shape), _rep(sc1b.shape), _rep(sh1b.shape),
            _rep(m2.shape), _rep(sc2b.shape), _rep(sh2b.shape),
            _rep(m3.shape), _rep(sc3b.shape), _rep(sh3b.shape),
        ],
        out_specs=pl.BlockSpec((8, B, 512), lambda b: (0, b, 0)),
        compiler_params=pltpu.CompilerParams(
            dimension_semantics=("parallel",)),
    )(xt, m1, sc1b, sh1b, m2, sc2b, sh2b, m3, sc3b, sh3b)

    f_conv3 = (fmap_raw.reshape(8, N, 8, 64).transpose(1, 3, 0, 2)
               .reshape(N, 4096))

    w1f = (fc1_w.reshape(64, 8, 8, 512).transpose(1, 2, 0, 3)
           .reshape(8, 512, 512).astype(bf16))
    w2f = fc2_w.astype(bf16)
    f_fc1, f_fc2 = pl.pallas_call(
        _fc_kernel,
        out_shape=(jax.ShapeDtypeStruct((N, 512), f32),
                   jax.ShapeDtypeStruct((N, 256), f32)),
        grid=(2,),
        in_specs=[
            pl.BlockSpec((8, N // 2, 512), lambda b: (0, b, 0)),
            _rep(w1f.shape), _rep(fc1_sc.shape), _rep(fc1_sh.shape),
            _rep(w2f.shape), _rep(fc2_sc.shape), _rep(fc2_sh.shape),
        ],
        out_specs=(pl.BlockSpec((N // 2, 512), lambda b: (b, 0)),
                   pl.BlockSpec((N // 2, 256), lambda b: (b, 0))),
        compiler_params=pltpu.CompilerParams(
            dimension_semantics=("parallel",)),
    )(fmap_raw, w1f, fc1_sc, fc1_sh, w2f, fc2_sc, fc2_sh)

    return {"f_conv3": f_conv3, "f_fc1": f_fc1, "f_fc2": f_fc2}


# speedup vs baseline: 23.6869x; 1.0070x over previous
import jax
import jax.numpy as jnp
from jax.experimental import pallas as pl
from jax.experimental.pallas import tpu as pltpu

_B = 32
_BH = 16


def _rep(shape):
    zeros = (0,) * len(shape)
    return pl.BlockSpec(shape, lambda *_: zeros)


def _pool_h(a):
    H = a.shape[0]
    r = a.reshape((H // 2, 2) + a.shape[1:])
    e, o = r[:, 0], r[:, 1]
    z = jnp.zeros_like(e[0:1])
    op = jnp.concatenate([z, o[:-1]], axis=0)
    en = jnp.concatenate([e[1:], z], axis=0)
    return jnp.maximum(jnp.maximum(e, o), jnp.maximum(op, en))


def _pool_w(a, step):
    ev, od = a[..., :512], a[..., 512:]
    z = jnp.zeros(a.shape[:-1] + (step,), a.dtype)
    odp = jnp.concatenate([z, od[..., :512 - step]], axis=-1)
    evn = jnp.concatenate([ev[..., step:], z], axis=-1)
    return jnp.maximum(jnp.maximum(ev, od), jnp.maximum(odp, evn))


def _trunk_half(xb, m1_ref, sc1_ref, sh1_ref, m2_ref, sc2_ref, sh2_ref,
                m3_ref, sc3_ref, sh3_ref):
    B = _BH
    bf16, f32 = jnp.bfloat16, jnp.float32

    zw = jnp.zeros((64, B, 2), bf16)
    xp = jnp.concatenate([zw, xb, zw], axis=2)
    zh = jnp.zeros((2, B, 68), bf16)
    xp = jnp.concatenate([zh, xp, zh], axis=0)
    x5 = jnp.concatenate([xp[kh:kh + 64] for kh in range(5)], axis=2)
    y1 = jnp.dot(x5.reshape(64 * B, 340), m1_ref[...],
                 preferred_element_type=f32)
    a1 = jnp.maximum(y1 * sc1_ref[...] + sh1_ref[...], 0.0).astype(bf16)
    a1 = a1.reshape(64, B, 1024)

    act1 = _pool_w(_pool_h(a1), 16)

    zw = jnp.zeros((32, B, 32), bf16)
    t = jnp.concatenate([zw, act1, zw], axis=2)
    zh = jnp.zeros((2, B, 576), bf16)
    a1p = jnp.concatenate([zh, t, zh], axis=0)
    yg = []
    for g in range(4):
        acc = jnp.dot(a1p[0:32, :, 128 * g:128 * g + 192].reshape(32 * B, 192),
                      m2_ref[0], preferred_element_type=f32)
        for kh in range(1, 5):
            acc = acc + jnp.dot(
                a1p[kh:kh + 32, :, 128 * g:128 * g + 192].reshape(32 * B, 192),
                m2_ref[kh], preferred_element_type=f32)
        yg.append(acc)
    y2 = jnp.concatenate([y[:, :128] for y in yg]
                         + [y[:, 128:] for y in yg], axis=1)
    a2 = jnp.maximum(y2 * sc2_ref[...] + sh2_ref[...], 0.0).astype(bf16)
    a2 = a2.reshape(32, B, 1024)

    act2 = _pool_w(_pool_h(a2), 32)

    zw = jnp.zeros((16, B, 64), bf16)
    t = jnp.concatenate([zw, act2, zw], axis=2)
    zh = jnp.zeros((2, B, 640), bf16)
    a2p = jnp.concatenate([zh, t, zh], axis=0)
    yg = []
    for g in range(4):
        acc = jnp.dot(a2p[0:16, :, 128 * g:128 * g + 256].reshape(16 * B, 256),
                      m3_ref[0], preferred_element_type=f32)
        for kh in range(1, 5):
            acc = acc + jnp.dot(
                a2p[kh:kh + 16, :, 128 * g:128 * g + 256].reshape(16 * B, 256),
                m3_ref[kh], preferred_element_type=f32)
        yg.append(acc)
    y3 = jnp.concatenate([y[:, :128] for y in yg]
                         + [y[:, 128:] for y in yg], axis=1)
    a3 = jnp.maximum(y3 * sc3_ref[...] + sh3_ref[...], 0.0).astype(bf16)
    a3 = a3.reshape(16, B, 1024)

    return _pool_w(_pool_h(a3), 64)


def _conv_trunk_kernel(x_ref, m1_ref, sc1_ref, sh1_ref, m2_ref, sc2_ref, sh2_ref,
                       m3_ref, sc3_ref, sh3_ref, fc3_ref):
    args = (m1_ref, sc1_ref, sh1_ref, m2_ref, sc2_ref, sh2_ref,
            m3_ref, sc3_ref, sh3_ref)
    fc3_ref[:, 0:_BH, :] = _trunk_half(x_ref[:, 0:_BH, :], *args)
    fc3_ref[:, _BH:_B, :] = _trunk_half(x_ref[:, _BH:_B, :], *args)


def _fc_kernel(x_ref, w1_ref, sc1_ref, sh1_ref, w2_ref, sc2_ref, sh2_ref,
               f1_ref, f2_ref):
    f32 = jnp.float32
    z1 = jnp.dot(x_ref[0], w1_ref[0], preferred_element_type=f32)
    for h in range(1, 8):
        z1 = z1 + jnp.dot(x_ref[h], w1_ref[h], preferred_element_type=f32)
    f1 = jnp.maximum(z1 * sc1_ref[...] + sh1_ref[...], 0.0)
    f1_ref[...] = f1
    z2 = jnp.dot(f1.astype(jnp.bfloat16), w2_ref[...], preferred_element_type=f32)
    f2_ref[...] = jnp.maximum(z2 * sc2_ref[...] + sh2_ref[...], 0.0)


def _band_weight(w_packed, K, Cin, Cout, Wout):
    Wpad = Wout + K - 1
    w4 = w_packed.reshape(K, K, Cin, Cout)
    eye = jnp.stack([jnp.eye(Wpad, Wout, -kw, dtype=w_packed.dtype)
                     for kw in range(K)])
    m = jnp.einsum('qvw,kqic->kviwc', eye, w4)
    m = m.reshape(K, Wpad, Cin, Wout // 2, 2, Cout)
    m = m.transpose(0, 1, 2, 4, 3, 5)
    return m.reshape(K, Wpad * Cin, Wout * Cout).astype(jnp.bfloat16)


def kernel(x, conv1_w, conv1_sc, conv1_sh, conv2_w, conv2_sc, conv2_sh,
           conv3_w, conv3_sc, conv3_sh, fc1_w, fc1_sc, fc1_sh,
           fc2_w, fc2_sc, fc2_sh):
    bf16, f32 = jnp.bfloat16, jnp.float32
    N = x.shape[0]
    B = _B
    xt = jnp.transpose(x.reshape(N, 64, 64).astype(bf16), (1, 0, 2))

    m1 = _band_weight(conv1_w, 5, 1, 16, 64).reshape(340, 1024)
    m2 = _band_weight(conv2_w, 5, 16, 32, 8)
    m3 = _band_weight(conv3_w, 5, 32, 64, 4)

    sc1b = jnp.tile(conv1_sc.reshape(16), 64).reshape(1, 1024)
    sh1b = jnp.tile(conv1_sh.reshape(16), 64).reshape(1, 1024)
    sc2b = jnp.tile(conv2_sc.reshape(32), 32).reshape(1, 1024)
    sh2b = jnp.tile(conv2_sh.reshape(32), 32).reshape(1, 1024)
    sc3b = jnp.tile(conv3_sc.reshape(64), 16).reshape(1, 1024)
    sh3b = jnp.tile(conv3_sh.reshape(64), 16).reshape(1, 1024)

    fmap_raw = pl.pallas_call(
        _conv_trunk_kernel,
        out_shape=jax.ShapeDtypeStruct((8, N, 512), bf16),
        grid=(N // B,),
        in_specs=[
            pl.BlockSpec((64, B, 64), lambda b: (0, b, 0)),
            _rep(m1.shape), _rep(sc1b.shape), _rep(sh1b.shape),
            _rep(m2.shape), _rep(sc2b.shape), _rep(sh2b.shape),
            _rep(m3.shape), _rep(sc3b.shape), _rep(sh3b.shape),
        ],
        out_specs=pl.BlockSpec((8, B, 512), lambda b: (0, b, 0)),
        compiler_params=pltpu.CompilerParams(
            dimension_semantics=("parallel",)),
    )(xt, m1, sc1b, sh1b, m2, sc2b, sh2b, m3, sc3b, sh3b)

    f_conv3 = (fmap_raw.reshape(8, N, 8, 64).transpose(1, 3, 0, 2)
               .reshape(N, 4096).astype(f32))

    w1f = (fc1_w.reshape(64, 8, 8, 512).transpose(1, 2, 0, 3)
           .reshape(8, 512, 512).astype(bf16))
    w2f = fc2_w.astype(bf16)
    f_fc1, f_fc2 = pl.pallas_call(
        _fc_kernel,
        out_shape=(jax.ShapeDtypeStruct((N, 512), f32),
                   jax.ShapeDtypeStruct((N, 256), f32)),
        grid=(2,),
        in_specs=[
            pl.BlockSpec((8, N // 2, 512), lambda b: (0, b, 0)),
            _rep(w1f.shape), _rep(fc1_sc.shape), _rep(fc1_sh.shape),
            _rep(w2f.shape), _rep(fc2_sc.shape), _rep(fc2_sh.shape),
        ],
        out_specs=(pl.BlockSpec((N // 2, 512), lambda b: (b, 0)),
                   pl.BlockSpec((N // 2, 256), lambda b: (b, 0))),
        compiler_params=pltpu.CompilerParams(
            dimension_semantics=("parallel",)),
    )(fmap_raw, w1f, fc1_sc, fc1_sh, w2f, fc2_sc, fc2_sh)

    return {"f_conv3": f_conv3, "f_fc1": f_fc1, "f_fc2": f_fc2}
